# baseline (speedup 1.0000x reference)
import functools

import jax
import jax.numpy as jnp
from jax.experimental import pallas as pl
from jax.experimental.pallas import tpu as pltpu

_C = 32768


def _blend(i, p_ref, kp_ref, q_ref, o_ref, c, b, kq):
    p = p_ref[0]
    s_raw = i * c - p
    s = jnp.where(s_raw < -c, s_raw + kq, s_raw)

    @pl.when(s >= b)
    def _copy():
        o_ref[...] = q_ref[...]

    @pl.when(s < b)
    def _window():
        d = q_ref.shape[0]
        u = s + c
        u128 = jnp.floor_divide(u, 128) * 128
        r = u - u128
        ext = kp_ref[:, pl.ds(pl.multiple_of(u128, 128), c + 128)]
        rolled = pltpu.roll(ext, (c + 128) - r, 1)
        shifted = rolled[:, :c]
        cols = jax.lax.broadcasted_iota(jnp.int32, (d, c), 1) + s
        mask = (cols >= 0) & (cols < b)
        o_ref[...] = jnp.where(mask, shifted, q_ref[...])


def _body(p1_ref, p2_ref, kp1_ref, kp2_ref, q1_ref, q2_ref,
          o1_ref, o2_ref, np1_ref, np2_ref, *, b, kq):
    i = pl.program_id(0)
    _blend(i, p1_ref, kp1_ref, q1_ref, o1_ref, _C, b, kq)
    _blend(i, p2_ref, kp2_ref, q2_ref, o2_ref, _C, b, kq)

    @pl.when(i == 0)
    def _ptrs():
        np1_ref[0] = jax.lax.rem(p1_ref[0] + b, kq)
        np2_ref[0] = jax.lax.rem(p2_ref[0] + b, kq)


def kernel(keys_1, keys_2, queue_1, queue_2, queue_1_ptr, queue_2_ptr):
    kq, d1 = queue_1.shape
    d2 = queue_2.shape[1]
    b = keys_1.shape[0]
    steps = pl.cdiv(kq, _C)

    q1t = queue_1.T
    q2t = queue_2.T
    kp1 = jnp.pad(keys_1.T, ((0, 0), (_C, _C + 128)))
    kp2 = jnp.pad(keys_2.T, ((0, 0), (_C, _C + 128)))

    out = pl.pallas_call(
        functools.partial(_body, b=b, kq=kq),
        grid=(steps,),
        in_specs=[
            pl.BlockSpec(memory_space=pltpu.SMEM),
            pl.BlockSpec(memory_space=pltpu.SMEM),
            pl.BlockSpec((d1, b + 2 * _C + 128), lambda i: (0, 0)),
            pl.BlockSpec((d2, b + 2 * _C + 128), lambda i: (0, 0)),
            pl.BlockSpec((d1, _C), lambda i: (0, i)),
            pl.BlockSpec((d2, _C), lambda i: (0, i)),
        ],
        out_specs=[
            pl.BlockSpec((d1, _C), lambda i: (0, i)),
            pl.BlockSpec((d2, _C), lambda i: (0, i)),
            pl.BlockSpec(memory_space=pltpu.SMEM),
            pl.BlockSpec(memory_space=pltpu.SMEM),
        ],
        out_shape=[
            jax.ShapeDtypeStruct((d1, kq), queue_1.dtype),
            jax.ShapeDtypeStruct((d2, kq), queue_2.dtype),
            jax.ShapeDtypeStruct((1,), jnp.int32),
            jax.ShapeDtypeStruct((1,), jnp.int32),
        ],
    )(queue_1_ptr, queue_2_ptr, kp1, kp2, q1t, q2t)
    return out[0].T, out[1].T, out[2], out[3]


# speedup vs baseline: 1.0155x; 1.0155x over previous
import functools

import jax
import jax.numpy as jnp
from jax.experimental import pallas as pl
from jax.experimental.pallas import tpu as pltpu

_C = 24576


def _blend(i, p_ref, kp_ref, q_ref, o_ref, c, b, kq):
    p = p_ref[0]
    s_raw = i * c - p
    s = jnp.where(s_raw < -c, s_raw + kq, s_raw)

    @pl.when(s >= b)
    def _copy():
        o_ref[...] = q_ref[...]

    @pl.when(s < b)
    def _window():
        d = q_ref.shape[0]
        u = s + c
        u128 = jnp.floor_divide(u, 128) * 128
        r = u - u128
        ext = kp_ref[:, pl.ds(pl.multiple_of(u128, 128), c + 128)]
        rolled = pltpu.roll(ext, (c + 128) - r, 1)
        shifted = rolled[:, :c]
        cols = jax.lax.broadcasted_iota(jnp.int32, (d, c), 1) + s
        mask = (cols >= 0) & (cols < b)
        o_ref[...] = jnp.where(mask, shifted, q_ref[...])


def _body(p1_ref, p2_ref, kp1_ref, kp2_ref, q1_ref, q2_ref,
          o1_ref, o2_ref, np1_ref, np2_ref, *, b, kq):
    i = pl.program_id(0)
    _blend(i, p1_ref, kp1_ref, q1_ref, o1_ref, _C, b, kq)
    _blend(i, p2_ref, kp2_ref, q2_ref, o2_ref, _C, b, kq)

    @pl.when(i == 0)
    def _ptrs():
        np1_ref[0] = jax.lax.rem(p1_ref[0] + b, kq)
        np2_ref[0] = jax.lax.rem(p2_ref[0] + b, kq)


def kernel(keys_1, keys_2, queue_1, queue_2, queue_1_ptr, queue_2_ptr):
    kq, d1 = queue_1.shape
    d2 = queue_2.shape[1]
    b = keys_1.shape[0]
    steps = pl.cdiv(kq, _C)

    q1t = queue_1.T
    q2t = queue_2.T
    kp1 = jnp.pad(keys_1.T, ((0, 0), (_C, _C + 128)))
    kp2 = jnp.pad(keys_2.T, ((0, 0), (_C, _C + 128)))

    out = pl.pallas_call(
        functools.partial(_body, b=b, kq=kq),
        grid=(steps,),
        in_specs=[
            pl.BlockSpec(memory_space=pltpu.SMEM),
            pl.BlockSpec(memory_space=pltpu.SMEM),
            pl.BlockSpec((d1, b + 2 * _C + 128), lambda i: (0, 0)),
            pl.BlockSpec((d2, b + 2 * _C + 128), lambda i: (0, 0)),
            pl.BlockSpec((d1, _C), lambda i: (0, i)),
            pl.BlockSpec((d2, _C), lambda i: (0, i)),
        ],
        out_specs=[
            pl.BlockSpec((d1, _C), lambda i: (0, i)),
            pl.BlockSpec((d2, _C), lambda i: (0, i)),
            pl.BlockSpec(memory_space=pltpu.SMEM),
            pl.BlockSpec(memory_space=pltpu.SMEM),
        ],
        out_shape=[
            jax.ShapeDtypeStruct((d1, kq), queue_1.dtype),
            jax.ShapeDtypeStruct((d2, kq), queue_2.dtype),
            jax.ShapeDtypeStruct((1,), jnp.int32),
            jax.ShapeDtypeStruct((1,), jnp.int32),
        ],
    )(queue_1_ptr, queue_2_ptr, kp1, kp2, q1t, q2t)
    return out[0].T, out[1].T, out[2], out[3]


# speedup vs baseline: 1.0173x; 1.0018x over previous
import functools

import jax
import jax.numpy as jnp
from jax.experimental import pallas as pl
from jax.experimental.pallas import tpu as pltpu

_C = 20480


def _blend(i, p_ref, kp_ref, q_ref, o_ref, c, b, kq):
    p = p_ref[0]
    s_raw = i * c - p
    s = jnp.where(s_raw < -c, s_raw + kq, s_raw)

    @pl.when(s >= b)
    def _copy():
        o_ref[...] = q_ref[...]

    @pl.when(s < b)
    def _window():
        d = q_ref.shape[0]
        u = s + c
        u128 = jnp.floor_divide(u, 128) * 128
        r = u - u128
        ext = kp_ref[:, pl.ds(pl.multiple_of(u128, 128), c + 128)]
        rolled = pltpu.roll(ext, (c + 128) - r, 1)
        shifted = rolled[:, :c]
        cols = jax.lax.broadcasted_iota(jnp.int32, (d, c), 1) + s
        mask = (cols >= 0) & (cols < b)
        o_ref[...] = jnp.where(mask, shifted, q_ref[...])


def _body(p1_ref, p2_ref, kp1_ref, kp2_ref, q1_ref, q2_ref,
          o1_ref, o2_ref, np1_ref, np2_ref, *, b, kq):
    i = pl.program_id(0)
    _blend(i, p1_ref, kp1_ref, q1_ref, o1_ref, _C, b, kq)
    _blend(i, p2_ref, kp2_ref, q2_ref, o2_ref, _C, b, kq)

    @pl.when(i == 0)
    def _ptrs():
        np1_ref[0] = jax.lax.rem(p1_ref[0] + b, kq)
        np2_ref[0] = jax.lax.rem(p2_ref[0] + b, kq)


def kernel(keys_1, keys_2, queue_1, queue_2, queue_1_ptr, queue_2_ptr):
    kq, d1 = queue_1.shape
    d2 = queue_2.shape[1]
    b = keys_1.shape[0]
    steps = pl.cdiv(kq, _C)

    q1t = queue_1.T
    q2t = queue_2.T
    kp1 = jnp.pad(keys_1.T, ((0, 0), (_C, _C + 128)))
    kp2 = jnp.pad(keys_2.T, ((0, 0), (_C, _C + 128)))

    out = pl.pallas_call(
        functools.partial(_body, b=b, kq=kq),
        grid=(steps,),
        in_specs=[
            pl.BlockSpec(memory_space=pltpu.SMEM),
            pl.BlockSpec(memory_space=pltpu.SMEM),
            pl.BlockSpec((d1, b + 2 * _C + 128), lambda i: (0, 0)),
            pl.BlockSpec((d2, b + 2 * _C + 128), lambda i: (0, 0)),
            pl.BlockSpec((d1, _C), lambda i: (0, i)),
            pl.BlockSpec((d2, _C), lambda i: (0, i)),
        ],
        out_specs=[
            pl.BlockSpec((d1, _C), lambda i: (0, i)),
            pl.BlockSpec((d2, _C), lambda i: (0, i)),
            pl.BlockSpec(memory_space=pltpu.SMEM),
            pl.BlockSpec(memory_space=pltpu.SMEM),
        ],
        out_shape=[
            jax.ShapeDtypeStruct((d1, kq), queue_1.dtype),
            jax.ShapeDtypeStruct((d2, kq), queue_2.dtype),
            jax.ShapeDtypeStruct((1,), jnp.int32),
            jax.ShapeDtypeStruct((1,), jnp.int32),
        ],
    )(queue_1_ptr, queue_2_ptr, kp1, kp2, q1t, q2t)
    return out[0].T, out[1].T, out[2], out[3]
